# Initial kernel scaffold; baseline (speedup 1.0000x reference)
import jax
import jax.numpy as jnp
from jax.experimental import pallas as pl


def kernel(weight, IDX, G):
    raise NotImplementedError("write your pallas kernel here")


# speedup vs baseline: 278.0625x; 278.0625x over previous
import functools

import jax
import jax.numpy as jnp
from jax import lax
from jax.experimental import pallas as pl
from jax.experimental.pallas import tpu as pltpu
from jax.experimental.pallas import tpu_sc as plsc

OUT_FEATURES = 4096
IN_FEATURES = 4096
TOTAL = OUT_FEATURES * IN_FEATURES
NUM_CORES = 2
NUM_SUBCORES = 16
NW = NUM_CORES * NUM_SUBCORES
PER_W = TOTAL // NW
CHUNK = 16384
NCHUNK = PER_W // CHUNK
LANES = 16

_mesh = plsc.VectorSubcoreMesh(core_axis_name="c", subcore_axis_name="s")


@functools.partial(
    pl.kernel,
    mesh=_mesh,
    out_type=jax.ShapeDtypeStruct((TOTAL,), jnp.float32),
    scratch_types=[
        pltpu.VMEM((CHUNK,), jnp.int32),
        pltpu.VMEM((CHUNK,), jnp.float32),
        pltpu.VMEM((CHUNK,), jnp.float32),
        pltpu.SemaphoreType.DMA,
    ],
)
def _ssl_gather(w_hbm, idx_hbm, g_hbm, out_hbm, idx_v, wv, gv, sem):
    wid = lax.axis_index("s") * NUM_CORES + lax.axis_index("c")
    base = wid * PER_W

    def chunk_body(i, _):
        off = base + i * CHUNK
        pltpu.sync_copy(idx_hbm.at[pl.ds(off, CHUNK)], idx_v)
        gather = pltpu.async_copy(w_hbm.at[idx_v], wv, sem)
        pltpu.sync_copy(g_hbm.at[pl.ds(off, CHUNK)], gv)
        gather.wait()

        def mul_body(j, _):
            s = pl.ds(pl.multiple_of(j * LANES, LANES), LANES)
            wv[s] = wv[s] * gv[s]
            return 0

        lax.fori_loop(0, CHUNK // LANES, mul_body, 0)
        pltpu.sync_copy(wv, out_hbm.at[pl.ds(off, CHUNK)])
        return 0

    lax.fori_loop(0, NCHUNK, chunk_body, 0)


def kernel(weight, IDX, G):
    idx_flat = jnp.ravel(IDX)
    g_flat = jnp.ravel(G)
    out = _ssl_gather(weight, idx_flat, g_flat)
    return out.reshape(OUT_FEATURES, IN_FEATURES)


# speedup vs baseline: 342.5930x; 1.2321x over previous
import functools

import jax
import jax.numpy as jnp
from jax import lax
from jax.experimental import pallas as pl
from jax.experimental.pallas import tpu as pltpu
from jax.experimental.pallas import tpu_sc as plsc

OUT_FEATURES = 4096
IN_FEATURES = 4096
TOTAL = OUT_FEATURES * IN_FEATURES
NUM_CORES = 2
NUM_SUBCORES = 16
NW = NUM_CORES * NUM_SUBCORES
PER_W = TOTAL // NW
CHUNK = 16384
NCHUNK = PER_W // CHUNK
LANES = 16
MUL_UNROLL = 8

_mesh = plsc.VectorSubcoreMesh(core_axis_name="c", subcore_axis_name="s")


@functools.partial(
    pl.kernel,
    mesh=_mesh,
    out_type=jax.ShapeDtypeStruct((TOTAL,), jnp.float32),
    scratch_types=[
        pltpu.VMEM((CHUNK,), jnp.int32),
        pltpu.VMEM((CHUNK,), jnp.int32),
        pltpu.VMEM((CHUNK,), jnp.float32),
        pltpu.VMEM((CHUNK,), jnp.float32),
        pltpu.VMEM((CHUNK,), jnp.float32),
        pltpu.VMEM((CHUNK,), jnp.float32),
        pltpu.SemaphoreType.DMA,
        pltpu.SemaphoreType.DMA,
    ],
)
def _ssl_gather(w_hbm, idx_hbm, g_hbm, out_hbm,
                idx0, idx1, wv0, wv1, gv0, gv1, sem0, sem1):
    wid = lax.axis_index("s") * NUM_CORES + lax.axis_index("c")
    base = wid * PER_W

    def multiply(wv, gv):
        def mul_body(j, _):
            for u in range(MUL_UNROLL):
                s = pl.ds(pl.multiple_of((j * MUL_UNROLL + u) * LANES, LANES),
                          LANES)
                wv[s] = wv[s] * gv[s]
            return 0

        lax.fori_loop(0, CHUNK // (LANES * MUL_UNROLL), mul_body, 0)

    pltpu.sync_copy(idx_hbm.at[pl.ds(base, CHUNK)], idx0)
    pltpu.async_copy(w_hbm.at[idx0], wv0, sem0)

    def pair_body(i2, _):
        i = i2 * 2
        off0 = base + i * CHUNK
        off1 = off0 + CHUNK

        pltpu.sync_copy(idx_hbm.at[pl.ds(off1, CHUNK)], idx1)
        pltpu.async_copy(w_hbm.at[idx1], wv1, sem1)

        pltpu.sync_copy(g_hbm.at[pl.ds(off0, CHUNK)], gv0)
        pltpu.make_async_copy(w_hbm.at[idx0], wv0, sem0).wait()
        multiply(wv0, gv0)
        pltpu.sync_copy(wv0, out_hbm.at[pl.ds(off0, CHUNK)])

        @pl.when(i2 + 1 < NCHUNK // 2)
        def _():
            off2 = off1 + CHUNK
            pltpu.sync_copy(idx_hbm.at[pl.ds(off2, CHUNK)], idx0)
            pltpu.async_copy(w_hbm.at[idx0], wv0, sem0)

        pltpu.sync_copy(g_hbm.at[pl.ds(off1, CHUNK)], gv1)
        pltpu.make_async_copy(w_hbm.at[idx1], wv1, sem1).wait()
        multiply(wv1, gv1)
        pltpu.sync_copy(wv1, out_hbm.at[pl.ds(off1, CHUNK)])
        return 0

    lax.fori_loop(0, NCHUNK // 2, pair_body, 0)


def kernel(weight, IDX, G):
    idx_flat = jnp.ravel(IDX)
    g_flat = jnp.ravel(G)
    out = _ssl_gather(weight, idx_flat, g_flat)
    return out.reshape(OUT_FEATURES, IN_FEATURES)


# speedup vs baseline: 343.0714x; 1.0014x over previous
import functools

import jax
import jax.numpy as jnp
from jax import lax
from jax.experimental import pallas as pl
from jax.experimental.pallas import tpu as pltpu
from jax.experimental.pallas import tpu_sc as plsc

OUT_FEATURES = 4096
IN_FEATURES = 4096
TOTAL = OUT_FEATURES * IN_FEATURES
NUM_CORES = 2
NUM_SUBCORES = 16
NW = NUM_CORES * NUM_SUBCORES
PER_W = TOTAL // NW
CHUNK = 16384
NCHUNK = PER_W // CHUNK
NC2 = NCHUNK // 2
LANES = 16
MUL_UNROLL = 8

_mesh = plsc.VectorSubcoreMesh(core_axis_name="c", subcore_axis_name="s")


@functools.partial(
    pl.kernel,
    mesh=_mesh,
    out_type=jax.ShapeDtypeStruct((TOTAL,), jnp.float32),
    scratch_types=[
        pltpu.VMEM((CHUNK,), jnp.int32),
        pltpu.VMEM((CHUNK,), jnp.int32),
        pltpu.VMEM((CHUNK,), jnp.float32),
        pltpu.VMEM((CHUNK,), jnp.float32),
        pltpu.VMEM((CHUNK,), jnp.float32),
        pltpu.VMEM((CHUNK,), jnp.float32),
        pltpu.SemaphoreType.DMA,
        pltpu.SemaphoreType.DMA,
        pltpu.SemaphoreType.DMA,
        pltpu.SemaphoreType.DMA,
        pltpu.SemaphoreType.DMA,
        pltpu.SemaphoreType.DMA,
        pltpu.SemaphoreType.DMA,
        pltpu.SemaphoreType.DMA,
    ],
)
def _ssl_gather(w_hbm, idx_hbm, g_hbm, out_hbm,
                idx0, idx1, wv0, wv1, gv0, gv1,
                si0, si1, sw0, sw1, sg0, sg1, so0, so1):
    wid = lax.axis_index("s") * NUM_CORES + lax.axis_index("c")
    base = wid * PER_W

    def multiply(wv, gv):
        def mul_body(j, _):
            for u in range(MUL_UNROLL):
                s = pl.ds(pl.multiple_of((j * MUL_UNROLL + u) * LANES, LANES),
                          LANES)
                wv[s] = wv[s] * gv[s]
            return 0

        lax.fori_loop(0, CHUNK // (LANES * MUL_UNROLL), mul_body, 0)

    def idx_slice(i):
        return idx_hbm.at[pl.ds(base + i * CHUNK, CHUNK)]

    def g_slice(i):
        return g_hbm.at[pl.ds(base + i * CHUNK, CHUNK)]

    def out_slice(i):
        return out_hbm.at[pl.ds(base + i * CHUNK, CHUNK)]

    pltpu.sync_copy(idx_slice(0), idx0)
    pltpu.async_copy(w_hbm.at[idx0], wv0, sw0)
    pltpu.async_copy(g_slice(0), gv0, sg0)
    pltpu.async_copy(g_slice(1), gv1, sg1)
    pltpu.async_copy(idx_slice(1), idx1, si1)

    def half(i, i2, idxa, idxb, wva, wvb, gva, sia, sib, swa, swb, sga, soa,
             sob, first, last_pair):
        if first:
            @pl.when(i2 > 0)
            def _():
                pltpu.make_async_copy(wvb, out_slice(i - 1), sob).wait()
        else:
            pltpu.make_async_copy(wvb, out_slice(i - 1), sob).wait()

        def fire_next_gather():
            pltpu.make_async_copy(idx_slice(i + 1), idxb, sib).wait()
            pltpu.async_copy(w_hbm.at[idxb], wvb, swb)

        if last_pair is None:
            fire_next_gather()
        else:
            pl.when(i2 < NC2 - 1)(fire_next_gather)

        pltpu.make_async_copy(w_hbm.at[idxa], wva, swa).wait()

        @pl.when(i2 < NC2 - 1)
        def _():
            pltpu.async_copy(idx_slice(i + 2), idxa, sia)

        pltpu.make_async_copy(g_slice(i), gva, sga).wait()
        multiply(wva, gva)
        pltpu.async_copy(wva, out_slice(i), soa)

        @pl.when(i2 < NC2 - 1)
        def _():
            pltpu.async_copy(g_slice(i + 2), gva, sga)

    def pair_body(i2, _):
        i = i2 * 2
        half(i, i2, idx0, idx1, wv0, wv1, gv0, si0, si1, sw0, sw1, sg0,
             so0, so1, first=True, last_pair=None)
        half(i + 1, i2, idx1, idx0, wv1, wv0, gv1, si1, si0, sw1, sw0, sg1,
             so1, so0, first=False, last_pair=True)
        return 0

    lax.fori_loop(0, NC2, pair_body, 0)

    pltpu.make_async_copy(wv1, out_slice(NCHUNK - 1), so1).wait()


def kernel(weight, IDX, G):
    idx_flat = jnp.ravel(IDX)
    g_flat = jnp.ravel(G)
    out = _ssl_gather(weight, idx_flat, g_flat)
    return out.reshape(OUT_FEATURES, IN_FEATURES)


# speedup vs baseline: 420.1870x; 1.2248x over previous
import functools

import jax
import jax.numpy as jnp
from jax import lax
from jax.experimental import pallas as pl
from jax.experimental.pallas import tpu as pltpu
from jax.experimental.pallas import tpu_sc as plsc

OUT_FEATURES = 4096
IN_FEATURES = 4096
NUM_CORES = 2
NUM_SUBCORES = 16
NW = NUM_CORES * NUM_SUBCORES
ROWS_PER_W = OUT_FEATURES // NW
CROWS = 8
CCOLS = 1024
CHUNK = CROWS * CCOLS
COL_SLABS = IN_FEATURES // CCOLS
NCHUNK = (ROWS_PER_W // CROWS) * COL_SLABS
NC2 = NCHUNK // 2
LANES = 16
UNROLL = 8

_mesh = plsc.VectorSubcoreMesh(core_axis_name="c", subcore_axis_name="s")


@functools.partial(
    pl.kernel,
    mesh=_mesh,
    out_type=jax.ShapeDtypeStruct((OUT_FEATURES, IN_FEATURES), jnp.float32),
    compiler_params=pltpu.CompilerParams(use_tc_tiling_on_sc=True),
    scratch_types=[
        pltpu.VMEM((CROWS, CCOLS), jnp.int32),
        pltpu.VMEM((CROWS, CCOLS), jnp.int32),
        pltpu.VMEM((CHUNK,), jnp.int32),
        pltpu.VMEM((CHUNK,), jnp.int32),
        pltpu.VMEM((CHUNK,), jnp.float32),
        pltpu.VMEM((CHUNK,), jnp.float32),
        pltpu.VMEM((CROWS, CCOLS), jnp.float32),
        pltpu.VMEM((CROWS, CCOLS), jnp.float32),
        pltpu.VMEM((CROWS, CCOLS), jnp.float32),
        pltpu.VMEM((CROWS, CCOLS), jnp.float32),
        pltpu.SemaphoreType.DMA,
        pltpu.SemaphoreType.DMA,
        pltpu.SemaphoreType.DMA,
        pltpu.SemaphoreType.DMA,
        pltpu.SemaphoreType.DMA,
        pltpu.SemaphoreType.DMA,
        pltpu.SemaphoreType.DMA,
        pltpu.SemaphoreType.DMA,
    ],
)
def _ssl_gather(w_hbm, idx_hbm, g_hbm, out_hbm,
                ib0, ib1, il0, il1, wv0, wv1, gb0, gb1, ob0, ob1,
                si0, si1, sw0, sw1, sg0, sg1, so0, so1):
    wid = lax.axis_index("s") * NUM_CORES + lax.axis_index("c")
    base_row = wid * ROWS_PER_W

    def relayout(ib, il):
        for r in range(CROWS):
            def body(j, _):
                for u in range(UNROLL):
                    c = pl.ds(
                        pl.multiple_of((j * UNROLL + u) * LANES, LANES),
                        LANES)
                    p = pl.ds(
                        pl.multiple_of(r * CCOLS + (j * UNROLL + u) * LANES,
                                       LANES), LANES)
                    il[p] = ib[r, c]
                return 0

            lax.fori_loop(0, CCOLS // (LANES * UNROLL), body, 0)

    def multiply(wv, gb, ob):
        for r in range(CROWS):
            def body(j, _):
                for u in range(UNROLL):
                    c = pl.ds(
                        pl.multiple_of((j * UNROLL + u) * LANES, LANES),
                        LANES)
                    p = pl.ds(
                        pl.multiple_of(r * CCOLS + (j * UNROLL + u) * LANES,
                                       LANES), LANES)
                    ob[r, c] = wv[p] * gb[r, c]
                return 0

            lax.fori_loop(0, CCOLS // (LANES * UNROLL), body, 0)

    def chunk_slice(ref, i):
        row = base_row + (i // COL_SLABS) * CROWS
        col = (i % COL_SLABS) * CCOLS
        return ref.at[pl.ds(row, CROWS), pl.ds(col, CCOLS)]

    pltpu.sync_copy(chunk_slice(idx_hbm, 0), ib0)
    relayout(ib0, il0)
    pltpu.async_copy(w_hbm.at[il0], wv0, sw0)
    pltpu.async_copy(chunk_slice(idx_hbm, 1), ib1, si1)
    pltpu.async_copy(chunk_slice(g_hbm, 0), gb0, sg0)
    pltpu.async_copy(chunk_slice(g_hbm, 1), gb1, sg1)

    def half(i, i2, iba, ibb, ila, ilb, wva, wvb, gba, oba,
             sia, sib, swa, swb, sga, soa, first):
        last_pair = i2 == NC2 - 1

        def fire_next():
            pltpu.make_async_copy(chunk_slice(idx_hbm, i + 1), ibb,
                                  sib).wait()
            relayout(ibb, ilb)
            pltpu.async_copy(w_hbm.at[ilb], wvb, swb)

        if first:
            fire_next()
        else:
            pl.when(i2 < NC2 - 1)(fire_next)

        @pl.when(i2 < NC2 - 1)
        def _():
            pltpu.async_copy(chunk_slice(idx_hbm, i + 2), iba, sia)

        pltpu.make_async_copy(w_hbm.at[ila], wva, swa).wait()
        pltpu.make_async_copy(chunk_slice(g_hbm, i), gba, sga).wait()
        if first:
            @pl.when(i2 > 0)
            def _():
                pltpu.make_async_copy(oba, chunk_slice(out_hbm, i - 2),
                                      soa).wait()
        else:
            @pl.when(i2 > 0)
            def _():
                pltpu.make_async_copy(oba, chunk_slice(out_hbm, i - 2),
                                      soa).wait()

        multiply(wva, gba, oba)
        pltpu.async_copy(oba, chunk_slice(out_hbm, i), soa)

        @pl.when(i2 < NC2 - 1)
        def _():
            pltpu.async_copy(chunk_slice(g_hbm, i + 2), gba, sga)

    def pair_body(i2, _):
        i = i2 * 2
        half(i, i2, ib0, ib1, il0, il1, wv0, wv1, gb0, ob0,
             si0, si1, sw0, sw1, sg0, so0, first=True)
        half(i + 1, i2, ib1, ib0, il1, il0, wv1, wv0, gb1, ob1,
             si1, si0, sw1, sw0, sg1, so1, first=False)
        return 0

    lax.fori_loop(0, NC2, pair_body, 0)

    pltpu.make_async_copy(ob0, chunk_slice(out_hbm, NCHUNK - 2), so0).wait()
    pltpu.make_async_copy(ob1, chunk_slice(out_hbm, NCHUNK - 1), so1).wait()


def kernel(weight, IDX, G):
    return _ssl_gather(weight, IDX, G)
